# baseline (speedup 1.0000x reference)
import math

import jax
import jax.numpy as jnp
from jax import lax
from jax.experimental import pallas as pl
from jax.experimental.pallas import tpu as pltpu
from jax.experimental.pallas import tpu_sc as plsc

VOCAB = 1000000
EMB = 64
B = 4096
L = 200
SCALE = math.sqrt(EMB)

NC = 2
NS = 16
NW = NC * NS
LANES = 16

BTILE = 128
OPAD = 137
NBUF = 4
NGROUP = L // NBUF


def _body(table_hbm, idx_hbm, out_hbm, idx_v, in_v, out_v, gsems, ssems):
  wid = lax.axis_index("s") * NC + lax.axis_index("c")
  b0 = wid * BTILE
  pltpu.sync_copy(idx_hbm.at[:, pl.ds(b0, BTILE)], idx_v)

  def gather_start(b, l):
    pltpu.async_copy(table_hbm.at[idx_v.at[l]], in_v.at[b], gsems[b])

  def gather_wait(b):
    pltpu.make_async_copy(table_hbm.at[idx_v.at[0]], in_v.at[b], gsems[b]).wait()

  def store_start(b, l):
    pltpu.async_copy(
        out_v.at[b, :, :, pl.ds(0, BTILE)], out_hbm.at[l, :, wid], ssems[b])

  def store_wait(b):
    pltpu.make_async_copy(
        out_v.at[b, :, :, pl.ds(0, BTILE)], out_hbm.at[0, :, wid], ssems[b]
    ).wait()

  lane = lax.iota(jnp.int32, LANES)
  st_lo = lane >> 3
  s_vec = lane & 7

  for b in range(NBUF):
    gather_start(b, b)

  def group_body(g, carry):
    for b in range(NBUF):
      l = g * NBUF + b
      gather_wait(b)
      @pl.when(g > 0)
      def _():
        store_wait(b)

      @plsc.parallel_loop(0, BTILE, step=1, unroll=8)
      def xpose_body(tok):
        col = jnp.full((LANES,), tok, jnp.int32)
        for e0 in range(0, EMB, LANES):
          v = in_v[b, tok, pl.ds(e0, LANES)]
          plsc.store_scatter(
              out_v.at[b], [st_lo + (e0 // 8), s_vec, col], v * SCALE)

      @pl.when(l + NBUF < L)
      def _():
        gather_start(b, l + NBUF)

      store_start(b, l)
    return carry

  lax.fori_loop(0, NGROUP, group_body, 0)

  for b in range(NBUF):
    store_wait(b)


@jax.jit
def _embed(tokens, embedding_weight):
  tok_t = jnp.transpose(tokens.astype(jnp.int32))
  mesh = plsc.VectorSubcoreMesh(core_axis_name="c", subcore_axis_name="s")
  tbl128 = jnp.pad(embedding_weight, ((0, 0), (0, EMB)))
  kfn = pl.kernel(
      _body,
      out_type=jax.ShapeDtypeStruct((L, EMB // 8, NW, 8, BTILE), jnp.float32),
      mesh=mesh,
      scratch_types=[
          pltpu.VMEM((L, BTILE), jnp.int32),
          pltpu.VMEM((NBUF, BTILE, 2 * EMB), jnp.float32),
          pltpu.VMEM((NBUF, EMB // 8, 8, OPAD), jnp.float32),
          [pltpu.SemaphoreType.DMA] * NBUF,
          [pltpu.SemaphoreType.DMA] * NBUF,
      ],
      compiler_params=pltpu.CompilerParams(
          use_tc_tiling_on_sc=False, needs_layout_passes=False),
  )
  out5 = kfn(tbl128, tok_t)
  out = jnp.transpose(out5, (2, 4, 0, 1, 3))
  return jnp.reshape(out, (B, L, EMB))


def kernel(tokens, embedding_weight):
  return _embed(tokens, embedding_weight)


# speedup vs baseline: 1.0101x; 1.0101x over previous
import math

import jax
import jax.numpy as jnp
from jax import lax
from jax.experimental import pallas as pl
from jax.experimental.pallas import tpu as pltpu
from jax.experimental.pallas import tpu_sc as plsc

VOCAB = 1000000
EMB = 64
B = 4096
L = 200
SCALE = math.sqrt(EMB)

NC = 2
NS = 16
NW = NC * NS
LANES = 16

BTILE = 128
OPAD = 137
NBUF = 4
NGROUP = L // NBUF


def _body(table_hbm, idx_hbm, out_hbm, idx_v, in_v, out_v, gsems, ssems):
  wid = lax.axis_index("s") * NC + lax.axis_index("c")
  b0 = wid * BTILE
  pltpu.sync_copy(idx_hbm.at[:, pl.ds(b0, BTILE)], idx_v)

  def gather_start(b, l):
    pltpu.async_copy(table_hbm.at[idx_v.at[l]], in_v.at[b], gsems[b])

  def gather_wait(b):
    pltpu.make_async_copy(table_hbm.at[idx_v.at[0]], in_v.at[b], gsems[b]).wait()

  def store_start(b, l):
    pltpu.async_copy(
        out_v.at[b, :, :, pl.ds(0, BTILE)], out_hbm.at[l, :, wid], ssems[b])

  def store_wait(b):
    pltpu.make_async_copy(
        out_v.at[b, :, :, pl.ds(0, BTILE)], out_hbm.at[0, :, wid], ssems[b]
    ).wait()

  lane = lax.iota(jnp.int32, LANES)
  st_lo = lane >> 3
  s_vec = lane & 7

  for b in range(NBUF):
    gather_start(b, b)

  def group_body(g, carry):
    for b in range(NBUF):
      l = g * NBUF + b
      gather_wait(b)
      @pl.when(g > 0)
      def _():
        store_wait(b)

      @plsc.parallel_loop(0, BTILE, step=1, unroll=16)
      def xpose_body(tok):
        col = jnp.full((LANES,), tok, jnp.int32)
        for e0 in range(0, EMB, LANES):
          v = in_v[b, tok, pl.ds(e0, LANES)]
          plsc.store_scatter(
              out_v.at[b], [st_lo + (e0 // 8), s_vec, col], v * SCALE)

      @pl.when(l + NBUF < L)
      def _():
        gather_start(b, l + NBUF)

      store_start(b, l)
    return carry

  lax.fori_loop(0, NGROUP, group_body, 0)

  for b in range(NBUF):
    store_wait(b)


@jax.jit
def _embed(tokens, embedding_weight):
  tok_t = jnp.transpose(tokens.astype(jnp.int32))
  mesh = plsc.VectorSubcoreMesh(core_axis_name="c", subcore_axis_name="s")
  kfn = pl.kernel(
      _body,
      out_type=jax.ShapeDtypeStruct((L, EMB // 8, NW, 8, BTILE), jnp.float32),
      mesh=mesh,
      scratch_types=[
          pltpu.VMEM((L, BTILE), jnp.int32),
          pltpu.VMEM((NBUF, BTILE, EMB), jnp.float32),
          pltpu.VMEM((NBUF, EMB // 8, 8, OPAD), jnp.float32),
          [pltpu.SemaphoreType.DMA] * NBUF,
          [pltpu.SemaphoreType.DMA] * NBUF,
      ],
      compiler_params=pltpu.CompilerParams(
          use_tc_tiling_on_sc=False, needs_layout_passes=False),
  )
  out5 = kfn(embedding_weight, tok_t)
  out = jnp.transpose(out5, (2, 4, 0, 1, 3))
  return jnp.reshape(out, (B, L, EMB))


def kernel(tokens, embedding_weight):
  return _embed(tokens, embedding_weight)


# speedup vs baseline: 1.0142x; 1.0040x over previous
import math

import jax
import jax.numpy as jnp
from jax import lax
from jax.experimental import pallas as pl
from jax.experimental.pallas import tpu as pltpu
from jax.experimental.pallas import tpu_sc as plsc

VOCAB = 1000000
EMB = 64
B = 4096
L = 200
SCALE = math.sqrt(EMB)

NC = 2
NS = 16
NW = NC * NS
LANES = 16

BTILE = 128
OPAD = 137
NBUF = 4
NGROUP = L // NBUF


def _body(table_hbm, idx_hbm, out_hbm, idx_v, in_v, out_v, gsems, ssems):
  wid = lax.axis_index("s") * NC + lax.axis_index("c")
  b0 = wid * BTILE
  pltpu.sync_copy(idx_hbm.at[:, pl.ds(b0, BTILE)], idx_v)

  def gather_start(b, l):
    pltpu.async_copy(table_hbm.at[idx_v.at[l]], in_v.at[b], gsems[b])

  def gather_wait(b):
    pltpu.make_async_copy(table_hbm.at[idx_v.at[0]], in_v.at[b], gsems[b]).wait()

  def store_start(b, l):
    pltpu.async_copy(
        out_v.at[b, :, :, pl.ds(0, BTILE)], out_hbm.at[l, :, wid], ssems[b])

  def store_wait(b):
    pltpu.make_async_copy(
        out_v.at[b, :, :, pl.ds(0, BTILE)], out_hbm.at[0, :, wid], ssems[b]
    ).wait()

  lane = lax.iota(jnp.int32, LANES)
  st_lo = lane >> 3
  s_vec = lane & 7

  for b in range(NBUF):
    gather_start(b, b)

  def group_body(g, carry):
    for b in range(NBUF):
      l = g * NBUF + b
      gather_wait(b)
      @pl.when(g > 0)
      def _():
        store_wait(b)

      @plsc.parallel_loop(0, BTILE, step=1, unroll=8)
      def xpose_body(tok):
        col = jnp.full((LANES,), tok, jnp.int32)
        for e0 in range(0, EMB, LANES):
          v = in_v[b, tok, pl.ds(e0, LANES)]
          plsc.store_scatter(
              out_v.at[b], [st_lo + (e0 // 8), s_vec, col], v * SCALE)

      @pl.when(l + NBUF < L)
      def _():
        gather_start(b, l + NBUF)

      store_start(b, l)
    return carry

  lax.fori_loop(0, NGROUP, group_body, 0)

  for b in range(NBUF):
    store_wait(b)


@jax.jit
def _embed(tokens, embedding_weight):
  tok_t = jnp.transpose(tokens.astype(jnp.int32))
  mesh = plsc.VectorSubcoreMesh(core_axis_name="c", subcore_axis_name="s")
  kfn = pl.kernel(
      _body,
      out_type=jax.ShapeDtypeStruct((L, EMB // 8, NW, 8, BTILE), jnp.float32),
      mesh=mesh,
      scratch_types=[
          pltpu.VMEM((L, BTILE), jnp.int32),
          pltpu.VMEM((NBUF, BTILE, EMB), jnp.float32),
          pltpu.VMEM((NBUF, EMB // 8, 8, OPAD), jnp.float32),
          [pltpu.SemaphoreType.DMA] * NBUF,
          [pltpu.SemaphoreType.DMA] * NBUF,
      ],
      compiler_params=pltpu.CompilerParams(
          use_tc_tiling_on_sc=False, needs_layout_passes=False),
  )
  out5 = kfn(embedding_weight, tok_t)
  out = jnp.transpose(out5, (2, 4, 0, 1, 3))
  return jnp.reshape(out, (B, L, EMB))


def kernel(tokens, embedding_weight):
  return _embed(tokens, embedding_weight)
